# baseline (device time: 21343 ns/iter reference)
import jax
import jax.numpy as jnp
from jax import lax
from jax.experimental import pallas as pl
from jax.experimental.pallas import tpu as pltpu

K = 16
N_PARTS = 8
IDX_MASK = 0xFFF
KEY_MASK = ~0xFFF
NEG = -(2 ** 31)
SIGN_FIX = 0x7FFFFFFF


def _to_keys(vals_f32, col_iota):
    bits = lax.bitcast_convert_type(vals_f32, jnp.int32)
    mkey = jnp.where(bits >= 0, bits, bits ^ SIGN_FIX)
    return (mkey & KEY_MASK) | col_iota


def _from_key(key):
    mk = key & KEY_MASK
    bits = jnp.where(mk >= 0, mk, mk ^ SIGN_FIX)
    return lax.bitcast_convert_type(bits, jnp.float32)


def _topk(keys):
    rows = keys.shape[0]
    out_pos = lax.broadcasted_iota(jnp.int32, (rows, K), 1)
    mx = jnp.max(keys, axis=1, keepdims=True)
    acc = jnp.broadcast_to(mx, (rows, K))
    for j in range(1, K):
        mx = jnp.max(jnp.where(keys < mx, keys, NEG), axis=1, keepdims=True)
        acc = jnp.where(out_pos == j, mx, acc)
    return acc


def kernel(x):
    m, n = x.shape
    rows_per = m // N_PARTS

    def body(x_ref, o_ref, cand_ref, xblk_ref, copy_sem, send_sems, recv_sems):
        my_x = lax.axis_index("x")
        my_y = lax.axis_index("y")
        my_z = lax.axis_index("z")
        p = my_x * 4 + my_z

        blk_copy = pltpu.make_async_copy(
            x_ref.at[pl.ds(p * rows_per, rows_per), :], xblk_ref, copy_sem
        )
        blk_copy.start()

        def coords(t, q):
            return (q // 4, my_y if t == 0 else 1 - my_y, q % 4)

        barrier = pltpu.get_barrier_semaphore()
        for t in (0, 1):
            for q in range(N_PARTS):
                pl.semaphore_signal(
                    barrier, inc=1, device_id=coords(t, q),
                    device_id_type=pl.DeviceIdType.MESH,
                )

        blk_copy.wait()
        cols = lax.broadcasted_iota(jnp.int32, (rows_per, n), 1)
        cand_ref[0, pl.ds(p * rows_per, rows_per), :] = _topk(
            _to_keys(xblk_ref[...], cols)
        )

        pl.semaphore_wait(barrier, 2 * N_PARTS)

        my_rows = pl.ds(p * rows_per, rows_per)
        for t in (0, 1):
            for q in range(N_PARTS):
                if t == 0:
                    send = pltpu.make_async_remote_copy(
                        src_ref=cand_ref.at[0, my_rows, :],
                        dst_ref=cand_ref.at[0, my_rows, :],
                        send_sem=send_sems.at[q],
                        recv_sem=recv_sems.at[p],
                        device_id=coords(0, q),
                        device_id_type=pl.DeviceIdType.MESH,
                    )
                    pl.when(q != p)(send.start)
                else:
                    send = pltpu.make_async_remote_copy(
                        src_ref=cand_ref.at[0, my_rows, :],
                        dst_ref=cand_ref.at[1, my_rows, :],
                        send_sem=send_sems.at[N_PARTS + q],
                        recv_sem=recv_sems.at[N_PARTS + p],
                        device_id=coords(1, q),
                        device_id_type=pl.DeviceIdType.MESH,
                    )
                    send.start()

        for t in (0, 1):
            for q in range(N_PARTS):
                done = pltpu.make_async_remote_copy(
                    src_ref=cand_ref.at[0, my_rows, :],
                    dst_ref=cand_ref.at[t, pl.ds(q * rows_per, rows_per), :],
                    send_sem=send_sems.at[t * N_PARTS + q],
                    recv_sem=recv_sems.at[t * N_PARTS + q],
                    device_id=coords(t, q),
                    device_id_type=pl.DeviceIdType.MESH,
                )
                if t == 0:
                    pl.when(q != p)(done.wait)
                else:
                    done.wait()

        merged = jnp.concatenate([cand_ref[0], cand_ref[1]], axis=1)
        pos = lax.broadcasted_iota(jnp.int32, (m, 2 * K), 1)
        merged = (merged & KEY_MASK) | pos
        o_ref[...] = _from_key(_topk(merged))

    return pl.pallas_call(
        body,
        out_shape=jax.ShapeDtypeStruct((m, K), jnp.float32),
        in_specs=[pl.BlockSpec(memory_space=pl.ANY)],
        out_specs=pl.BlockSpec(memory_space=pltpu.VMEM),
        scratch_shapes=[
            pltpu.VMEM((2, m, K), jnp.int32),
            pltpu.VMEM((m // N_PARTS, n), jnp.float32),
            pltpu.SemaphoreType.DMA,
            pltpu.SemaphoreType.DMA((2 * N_PARTS,)),
            pltpu.SemaphoreType.DMA((2 * N_PARTS,)),
        ],
        compiler_params=pltpu.CompilerParams(collective_id=0),
    )(x)


# device time: 21023 ns/iter; 1.0152x vs baseline; 1.0152x over previous
import jax
import jax.numpy as jnp
from jax import lax
from jax.experimental import pallas as pl
from jax.experimental.pallas import tpu as pltpu

K = 16
N_PARTS = 8
IDX_MASK = 0xFFF
KEY_MASK = ~0xFFF
NEG = -(2 ** 31)
SIGN_FIX = 0x7FFFFFFF


def _to_keys(vals_f32, col_iota):
    bits = lax.bitcast_convert_type(vals_f32, jnp.int32)
    mkey = jnp.where(bits >= 0, bits, bits ^ SIGN_FIX)
    return (mkey & KEY_MASK) | col_iota


def _from_key(key):
    mk = key & KEY_MASK
    bits = jnp.where(mk >= 0, mk, mk ^ SIGN_FIX)
    return lax.bitcast_convert_type(bits, jnp.float32)


def _topk(keys):
    rows = keys.shape[0]
    out_pos = lax.broadcasted_iota(jnp.int32, (rows, K), 1)
    mx = jnp.max(keys, axis=1, keepdims=True)
    acc = jnp.broadcast_to(mx, (rows, K))
    for j in range(1, K):
        mx = jnp.max(jnp.where(keys < mx, keys, NEG), axis=1, keepdims=True)
        acc = jnp.where(out_pos == j, mx, acc)
    return acc


def kernel(x):
    m, n = x.shape
    rows_per = m // N_PARTS

    def body(x_ref, o_ref, cand_ref, xblk_ref, copy_sem, send_sems, recv_sems):
        my_x = lax.axis_index("x")
        my_y = lax.axis_index("y")
        my_z = lax.axis_index("z")
        p = my_x * 4 + my_z

        blk_copy = pltpu.make_async_copy(
            x_ref.at[pl.ds(p * rows_per, rows_per), :], xblk_ref, copy_sem
        )
        blk_copy.start()

        def coords(t, q):
            return (q // 4, my_y if t == 0 else 1 - my_y, q % 4)

        barrier = pltpu.get_barrier_semaphore()
        for t in (0, 1):
            for q in range(N_PARTS):
                pl.semaphore_signal(
                    barrier, inc=1, device_id=coords(t, q),
                    device_id_type=pl.DeviceIdType.MESH,
                )

        blk_copy.wait()
        cols = lax.broadcasted_iota(jnp.int32, (rows_per, n), 1)
        keys = _to_keys(xblk_ref[...], cols)
        groups = [keys[:, g * 128:(g + 1) * 128] for g in range(n // 128)]
        mx = groups[0]
        for g in groups[1:]:
            mx = jnp.maximum(mx, g)
        cands = [mx]
        for _ in range(4):
            masked = [jnp.where(g < mx, g, NEG) for g in groups]
            mx = masked[0]
            for g in masked[1:]:
                mx = jnp.maximum(mx, g)
            cands.append(mx)
        cand_ref[0, pl.ds(p * rows_per, rows_per), :] = _topk(
            jnp.concatenate(cands, axis=1)
        )

        pl.semaphore_wait(barrier, 2 * N_PARTS)

        my_rows = pl.ds(p * rows_per, rows_per)
        for t in (0, 1):
            for q in range(N_PARTS):
                if t == 0:
                    send = pltpu.make_async_remote_copy(
                        src_ref=cand_ref.at[0, my_rows, :],
                        dst_ref=cand_ref.at[0, my_rows, :],
                        send_sem=send_sems.at[q],
                        recv_sem=recv_sems.at[p],
                        device_id=coords(0, q),
                        device_id_type=pl.DeviceIdType.MESH,
                    )
                    pl.when(q != p)(send.start)
                else:
                    send = pltpu.make_async_remote_copy(
                        src_ref=cand_ref.at[0, my_rows, :],
                        dst_ref=cand_ref.at[1, my_rows, :],
                        send_sem=send_sems.at[N_PARTS + q],
                        recv_sem=recv_sems.at[N_PARTS + p],
                        device_id=coords(1, q),
                        device_id_type=pl.DeviceIdType.MESH,
                    )
                    send.start()

        for t in (0, 1):
            for q in range(N_PARTS):
                done = pltpu.make_async_remote_copy(
                    src_ref=cand_ref.at[0, my_rows, :],
                    dst_ref=cand_ref.at[t, pl.ds(q * rows_per, rows_per), :],
                    send_sem=send_sems.at[t * N_PARTS + q],
                    recv_sem=recv_sems.at[t * N_PARTS + q],
                    device_id=coords(t, q),
                    device_id_type=pl.DeviceIdType.MESH,
                )
                if t == 0:
                    pl.when(q != p)(done.wait)
                else:
                    done.wait()

        merged = jnp.concatenate([cand_ref[0], cand_ref[1]], axis=1)
        pos = lax.broadcasted_iota(jnp.int32, (m, 2 * K), 1)
        merged = (merged & KEY_MASK) | pos
        o_ref[...] = _from_key(_topk(merged))

    return pl.pallas_call(
        body,
        out_shape=jax.ShapeDtypeStruct((m, K), jnp.float32),
        in_specs=[pl.BlockSpec(memory_space=pl.ANY)],
        out_specs=pl.BlockSpec(memory_space=pltpu.VMEM),
        scratch_shapes=[
            pltpu.VMEM((2, m, K), jnp.int32),
            pltpu.VMEM((m // N_PARTS, n), jnp.float32),
            pltpu.SemaphoreType.DMA,
            pltpu.SemaphoreType.DMA((2 * N_PARTS,)),
            pltpu.SemaphoreType.DMA((2 * N_PARTS,)),
        ],
        compiler_params=pltpu.CompilerParams(collective_id=0),
    )(x)
